# Initial kernel scaffold; baseline (speedup 1.0000x reference)
import jax
import jax.numpy as jnp
from jax import lax
from jax.experimental import pallas as pl
from jax.experimental.pallas import tpu as pltpu
from jax.experimental.pallas import tpu_sc as plsc

F32 = jnp.float32

_N = 10000
_E = 320000
_DF = 128
_DE = 16
_DU = 32

_NC = 2
_NS = 16
_NW = _NC * _NS
_B = 80
_G = 5
_SB = _B * _G
_BLK_PER_TILE = _E // (_NW * _B)
_SUP_PER_TILE = _BLK_PER_TILE // _G
_ZROWS = _N // _NS


def _sc_edge_body(ps_hbm, pd_hbm, q_hbm, ef_hbm, src_hbm, dst_hbm,
                  oef_hbm, agg_hbm, deg_hbm, esum_hbm,
                  sidx, didx, gs, gd, mb, efb, ones_v, accb, zb,
                  agg_sh, deg_sh, sem_a, sem_b, sem_c, sem_d):
    c = lax.axis_index("c")
    s = lax.axis_index("s")
    tid = c * _NS + s

    pltpu.sync_copy(src_hbm.at[pl.ds(tid * _BLK_PER_TILE, _BLK_PER_TILE)], sidx)
    pltpu.sync_copy(dst_hbm.at[pl.ds(tid * _BLK_PER_TILE, _BLK_PER_TILE)], didx)

    onerow = jnp.ones((_DE,), F32)
    zrow = jnp.zeros((_DE,), F32)

    def _ones_fill(i, carry):
        ones_v[i] = onerow
        return carry

    lax.fori_loop(0, _B, _ones_fill, 0)

    def _zero_fill(i, carry):
        zb[i] = zrow
        return carry

    lax.fori_loop(0, _ZROWS, _zero_fill, 0)
    pltpu.sync_copy(zb, agg_sh.at[pl.ds(s * _ZROWS, _ZROWS)])
    pltpu.sync_copy(zb, deg_sh.at[pl.ds(s * _ZROWS, _ZROWS)])
    plsc.subcore_barrier()

    def _super(t, acc):
        row0 = (tid * _SUP_PER_TILE + t) * _SB
        cps = [pltpu.async_copy(q_hbm.at[pl.ds(row0, _SB)], mb, sem_a),
               pltpu.async_copy(ef_hbm.at[pl.ds(row0, _SB)], efb, sem_b)]
        for k in range(_G):
            j = t * _G + k
            cps.append(pltpu.async_copy(
                ps_hbm.at[sidx.at[j]], gs.at[pl.ds(k * _B, _B)], sem_c))
            cps.append(pltpu.async_copy(
                pd_hbm.at[didx.at[j]], gd.at[pl.ds(k * _B, _B)], sem_d))
        for cp in cps:
            cp.wait()

        def _row(i, a):
            m = mb[i] + gs[i] + gd[i]
            mb[i] = m
            efb[i] = efb[i] + m
            return a + m

        acc = lax.fori_loop(0, _SB, _row, acc)

        wps = [pltpu.async_copy(efb, oef_hbm.at[pl.ds(row0, _SB)], sem_b)]
        for k in range(_G):
            j = t * _G + k
            wps.append(pltpu.async_copy(
                mb.at[pl.ds(k * _B, _B)], agg_sh.at[didx.at[j]], sem_c,
                add=True))
            wps.append(pltpu.async_copy(
                ones_v, deg_sh.at[didx.at[j]], sem_d, add=True))
        for cp in wps:
            cp.wait()
        return acc

    acc = lax.fori_loop(0, _SUP_PER_TILE, _super, jnp.zeros((_DE,), F32))

    accb[...] = acc
    pltpu.sync_copy(accb, esum_hbm.at[tid])

    plsc.subcore_barrier()

    @pl.when(s == 0)
    def _copy_out():
        pltpu.sync_copy(agg_sh, agg_hbm.at[c])
        pltpu.sync_copy(deg_sh, deg_hbm.at[c])


_sc_mesh = plsc.VectorSubcoreMesh(
    core_axis_name="c", subcore_axis_name="s",
    num_cores=_NC, num_subcores=_NS)

_sc_edge = pl.kernel(
    _sc_edge_body,
    out_type=(
        jax.ShapeDtypeStruct((_E, _DE), F32),
        jax.ShapeDtypeStruct((_NC, _N, _DE), F32),
        jax.ShapeDtypeStruct((_NC, _N, _DE), F32),
        jax.ShapeDtypeStruct((_NW, _DE), F32),
    ),
    mesh=_sc_mesh,
    scratch_types=[
        pltpu.VMEM((_BLK_PER_TILE, _B), jnp.int32),
        pltpu.VMEM((_BLK_PER_TILE, _B), jnp.int32),
        pltpu.VMEM((_SB, _DE), F32),
        pltpu.VMEM((_SB, _DE), F32),
        pltpu.VMEM((_SB, _DE), F32),
        pltpu.VMEM((_SB, _DE), F32),
        pltpu.VMEM((_B, _DE), F32),
        pltpu.VMEM((_DE,), F32),
        pltpu.VMEM((_ZROWS, _DE), F32),
        pltpu.VMEM_SHARED((_N, _DE), F32),
        pltpu.VMEM_SHARED((_N, _DE), F32),
        pltpu.SemaphoreType.DMA,
        pltpu.SemaphoreType.DMA,
        pltpu.SemaphoreType.DMA,
        pltpu.SemaphoreType.DMA,
    ],
)


_NBLK = 400
_EBLK = 2000


def _proj_body(nf_ref, ws_ref, wd_ref, ps_ref, pd_ref):
    x = nf_ref[...]
    ps_ref[...] = jnp.dot(x, ws_ref[...], preferred_element_type=F32)
    pd_ref[...] = jnp.dot(x, wd_ref[...], preferred_element_type=F32)


def _q_body(ef_ref, wee_ref, u_ref, weu_ref, be_ref, q_ref):
    cst = jnp.dot(u_ref[...], weu_ref[...], preferred_element_type=F32) \
        + be_ref[...]
    q_ref[...] = jnp.dot(ef_ref[...], wee_ref[...],
                         preferred_element_type=F32) + cst


def _node_body(nf_ref, agg_ref, deg_ref, wa_ref, wn_ref, u_ref, wnu_ref,
               bn_ref, out_ref, ns_ref):
    agg = agg_ref[0] + agg_ref[1]
    deg = deg_ref[0] + deg_ref[1]
    aggm = agg / jnp.maximum(deg, 1.0)
    cst = jnp.dot(u_ref[...], wnu_ref[...], preferred_element_type=F32) \
        + bn_ref[...]
    pre = (jnp.dot(aggm, wa_ref[...], preferred_element_type=F32)
           + jnp.dot(nf_ref[...], wn_ref[...], preferred_element_type=F32)
           + cst)
    out_ref[...] = pre + nf_ref[...]

    @pl.when(pl.program_id(0) == 0)
    def _():
        ns_ref[...] = jnp.zeros_like(ns_ref)

    ns_ref[...] += jnp.sum(pre, axis=0, keepdims=True)


def _glob_body(ns_ref, es_ref, u_ref, wgn_ref, wge_ref, wgu_ref, bg_ref,
               ou_ref):
    nr = ns_ref[...] * (1.0 / _N)
    er = jnp.sum(es_ref[...], axis=0, keepdims=True) * (1.0 / _E)
    ou_ref[...] = (jnp.dot(nr, wgn_ref[...], preferred_element_type=F32)
                   + jnp.dot(er, wge_ref[...], preferred_element_type=F32)
                   + jnp.dot(u_ref[...], wgu_ref[...],
                             preferred_element_type=F32)
                   + bg_ref[...] + u_ref[...])


def kernel(nf, ef, u, edge_index, We, be, Wn, bn, Wg, bg):
    assert nf.shape == (_N, _DF) and ef.shape == (_E, _DE)
    assert u.shape == (1, _DU) and edge_index.shape == (2, _E)

    src = edge_index[0].astype(jnp.int32).reshape(_E // _B, _B)
    dst = edge_index[1].astype(jnp.int32).reshape(_E // _B, _B)

    We_s = We[:_DF]
    We_d = We[_DF:2 * _DF]
    We_e = We[2 * _DF:2 * _DF + _DE]
    We_u = We[2 * _DF + _DE:]
    Wn_a = Wn[:_DE]
    Wn_n = Wn[_DE:_DE + _DF]
    Wn_u = Wn[_DE + _DF:]
    Wg_n = Wg[:_DF]
    Wg_e = Wg[_DF:_DF + _DE]
    Wg_u = Wg[_DF + _DE:]
    be2 = be.reshape(1, _DE)
    bn2 = bn.reshape(1, _DF)
    bg2 = bg.reshape(1, _DU)

    ngrid = _N // _NBLK
    ps, pd = pl.pallas_call(
        _proj_body,
        grid=(ngrid,),
        in_specs=[
            pl.BlockSpec((_NBLK, _DF), lambda i: (i, 0)),
            pl.BlockSpec((_DF, _DE), lambda i: (0, 0)),
            pl.BlockSpec((_DF, _DE), lambda i: (0, 0)),
        ],
        out_specs=[
            pl.BlockSpec((_NBLK, _DE), lambda i: (i, 0)),
            pl.BlockSpec((_NBLK, _DE), lambda i: (i, 0)),
        ],
        out_shape=[
            jax.ShapeDtypeStruct((_N, _DE), F32),
            jax.ShapeDtypeStruct((_N, _DE), F32),
        ],
    )(nf, We_s, We_d)

    egrid = _E // _EBLK
    q = pl.pallas_call(
        _q_body,
        grid=(egrid,),
        in_specs=[
            pl.BlockSpec((_EBLK, _DE), lambda i: (i, 0)),
            pl.BlockSpec((_DE, _DE), lambda i: (0, 0)),
            pl.BlockSpec((1, _DU), lambda i: (0, 0)),
            pl.BlockSpec((_DU, _DE), lambda i: (0, 0)),
            pl.BlockSpec((1, _DE), lambda i: (0, 0)),
        ],
        out_specs=pl.BlockSpec((_EBLK, _DE), lambda i: (i, 0)),
        out_shape=jax.ShapeDtypeStruct((_E, _DE), F32),
    )(ef, We_e, u, We_u, be2)

    oef, aggp, degp, esum = _sc_edge(ps, pd, q, ef, src, dst)

    onf, nsum = pl.pallas_call(
        _node_body,
        grid=(ngrid,),
        in_specs=[
            pl.BlockSpec((_NBLK, _DF), lambda i: (i, 0)),
            pl.BlockSpec((_NC, _NBLK, _DE), lambda i: (0, i, 0)),
            pl.BlockSpec((_NC, _NBLK, _DE), lambda i: (0, i, 0)),
            pl.BlockSpec((_DE, _DF), lambda i: (0, 0)),
            pl.BlockSpec((_DF, _DF), lambda i: (0, 0)),
            pl.BlockSpec((1, _DU), lambda i: (0, 0)),
            pl.BlockSpec((_DU, _DF), lambda i: (0, 0)),
            pl.BlockSpec((1, _DF), lambda i: (0, 0)),
        ],
        out_specs=[
            pl.BlockSpec((_NBLK, _DF), lambda i: (i, 0)),
            pl.BlockSpec((1, _DF), lambda i: (0, 0)),
        ],
        out_shape=[
            jax.ShapeDtypeStruct((_N, _DF), F32),
            jax.ShapeDtypeStruct((1, _DF), F32),
        ],
    )(nf, aggp, degp, Wn_a, Wn_n, u, Wn_u, bn2)

    ou = pl.pallas_call(
        _glob_body,
        out_shape=jax.ShapeDtypeStruct((1, _DU), F32),
    )(nsum, esum, u, Wg_n, Wg_e, Wg_u, bg2)

    return onf, oef, ou


# speedup vs baseline: 4.1390x; 4.1390x over previous
import jax
import jax.numpy as jnp
from jax import lax
from jax.experimental import pallas as pl
from jax.experimental.pallas import tpu as pltpu
from jax.experimental.pallas import tpu_sc as plsc

F32 = jnp.float32

_N = 10000
_E = 320000
_DF = 128
_DE = 16
_DU = 32

_NC = 2
_NS = 16
_NW = _NC * _NS
_B = 80
_G = 5
_SB = _B * _G
_BLK_PER_TILE = _E // (_NW * _B)
_SUP_PER_TILE = _BLK_PER_TILE // _G
_ZROWS = _N // _NS


def _sc_edge_body(ps_hbm, pd_hbm, q_hbm, ef_hbm, src_hbm, dst_hbm,
                  oef_hbm, agg_hbm, deg_hbm, esum_hbm,
                  sidx, didx, gs, gd, mb, efb, ones_v, accb, zb,
                  agg_sh, deg_sh, sem_a, sem_b, sem_c, sem_d):
    c = lax.axis_index("c")
    s = lax.axis_index("s")
    tid = c * _NS + s

    pltpu.sync_copy(src_hbm.at[tid], sidx)
    pltpu.sync_copy(dst_hbm.at[tid], didx)

    onerow = jnp.ones((_DE,), F32)
    zrow = jnp.zeros((_DE,), F32)

    def _ones_fill(i, carry):
        ones_v[i] = onerow
        return carry

    lax.fori_loop(0, _B, _ones_fill, 0)

    def _zero_fill(i, carry):
        zb[i] = zrow
        return carry

    lax.fori_loop(0, _ZROWS, _zero_fill, 0)
    pltpu.sync_copy(zb, agg_sh.at[pl.ds(s * _ZROWS, _ZROWS)])
    pltpu.sync_copy(zb, deg_sh.at[pl.ds(s * _ZROWS, _ZROWS)])
    plsc.subcore_barrier()

    def _super(t, acc):
        row0 = (tid * _SUP_PER_TILE + t) * _SB
        cps = [pltpu.async_copy(q_hbm.at[pl.ds(row0, _SB)], mb, sem_a),
               pltpu.async_copy(ef_hbm.at[pl.ds(row0, _SB)], efb, sem_b)]
        for k in range(_G):
            j = t * _G + k
            cps.append(pltpu.async_copy(
                ps_hbm.at[sidx.at[j]], gs.at[pl.ds(k * _B, _B)], sem_c))
            cps.append(pltpu.async_copy(
                pd_hbm.at[didx.at[j]], gd.at[pl.ds(k * _B, _B)], sem_d))
        for cp in cps:
            cp.wait()

        def _row(i, a):
            m = mb[i] + gs[i] + gd[i]
            mb[i] = m
            efb[i] = efb[i] + m
            return a + m

        acc = lax.fori_loop(0, _SB, _row, acc)

        wps = [pltpu.async_copy(efb, oef_hbm.at[pl.ds(row0, _SB)], sem_b)]
        for k in range(_G):
            j = t * _G + k
            wps.append(pltpu.async_copy(
                mb.at[pl.ds(k * _B, _B)], agg_sh.at[didx.at[j]], sem_c,
                add=True))
            wps.append(pltpu.async_copy(
                ones_v, deg_sh.at[didx.at[j]], sem_d, add=True))
        for cp in wps:
            cp.wait()
        return acc

    acc = lax.fori_loop(0, _SUP_PER_TILE, _super, jnp.zeros((_DE,), F32))

    accb[0] = acc
    pltpu.sync_copy(accb, esum_hbm.at[tid])

    plsc.subcore_barrier()

    @pl.when(s == 0)
    def _copy_out():
        pltpu.sync_copy(agg_sh, agg_hbm.at[c])
        pltpu.sync_copy(deg_sh, deg_hbm.at[c])


_sc_mesh = plsc.VectorSubcoreMesh(
    core_axis_name="c", subcore_axis_name="s",
    num_cores=_NC, num_subcores=_NS)

_sc_edge = pl.kernel(
    _sc_edge_body,
    out_type=(
        jax.ShapeDtypeStruct((_E, _DE), F32),
        jax.ShapeDtypeStruct((_NC, _N, _DE), F32),
        jax.ShapeDtypeStruct((_NC, _N, _DE), F32),
        jax.ShapeDtypeStruct((_NW, 1, _DE), F32),
    ),
    mesh=_sc_mesh,
    scratch_types=[
        pltpu.VMEM((_BLK_PER_TILE, _B), jnp.int32),
        pltpu.VMEM((_BLK_PER_TILE, _B), jnp.int32),
        pltpu.VMEM((_SB, _DE), F32),
        pltpu.VMEM((_SB, _DE), F32),
        pltpu.VMEM((_SB, _DE), F32),
        pltpu.VMEM((_SB, _DE), F32),
        pltpu.VMEM((_B, _DE), F32),
        pltpu.VMEM((1, _DE), F32),
        pltpu.VMEM((_ZROWS, _DE), F32),
        pltpu.VMEM_SHARED((_N, _DE), F32),
        pltpu.VMEM_SHARED((_N, _DE), F32),
        pltpu.SemaphoreType.DMA,
        pltpu.SemaphoreType.DMA,
        pltpu.SemaphoreType.DMA,
        pltpu.SemaphoreType.DMA,
    ],
    compiler_params=pltpu.CompilerParams(use_tc_tiling_on_sc=False),
)


_NBLK = 400
_EBLK = 2000


def _proj_body(nf_ref, ws_ref, wd_ref, ps_ref, pd_ref):
    x = nf_ref[...]
    ps_ref[...] = jnp.dot(x, ws_ref[...], preferred_element_type=F32)
    pd_ref[...] = jnp.dot(x, wd_ref[...], preferred_element_type=F32)


def _q_body(ef_ref, wee_ref, u_ref, weu_ref, be_ref, q_ref):
    cst = jnp.dot(u_ref[...], weu_ref[...], preferred_element_type=F32) \
        + be_ref[...]
    q_ref[...] = jnp.dot(ef_ref[...], wee_ref[...],
                         preferred_element_type=F32) + cst


def _node_body(nf_ref, agg_ref, deg_ref, wa_ref, wn_ref, u_ref, wnu_ref,
               bn_ref, out_ref, ns_ref):
    agg = agg_ref[0] + agg_ref[1]
    deg = deg_ref[0] + deg_ref[1]
    aggm = agg / jnp.maximum(deg, 1.0)
    cst = jnp.dot(u_ref[...], wnu_ref[...], preferred_element_type=F32) \
        + bn_ref[...]
    pre = (jnp.dot(aggm, wa_ref[...], preferred_element_type=F32)
           + jnp.dot(nf_ref[...], wn_ref[...], preferred_element_type=F32)
           + cst)
    out_ref[...] = pre + nf_ref[...]

    @pl.when(pl.program_id(0) == 0)
    def _():
        ns_ref[...] = jnp.zeros_like(ns_ref)

    ns_ref[...] += jnp.sum(pre, axis=0, keepdims=True)


def _glob_body(ns_ref, es_ref, u_ref, wgn_ref, wge_ref, wgu_ref, bg_ref,
               ou_ref):
    nr = ns_ref[...] * (1.0 / _N)
    er = jnp.sum(es_ref[...], axis=0, keepdims=True) * (1.0 / _E)
    ou_ref[...] = (jnp.dot(nr, wgn_ref[...], preferred_element_type=F32)
                   + jnp.dot(er, wge_ref[...], preferred_element_type=F32)
                   + jnp.dot(u_ref[...], wgu_ref[...],
                             preferred_element_type=F32)
                   + bg_ref[...] + u_ref[...])


def kernel(nf, ef, u, edge_index, We, be, Wn, bn, Wg, bg):
    assert nf.shape == (_N, _DF) and ef.shape == (_E, _DE)
    assert u.shape == (1, _DU) and edge_index.shape == (2, _E)

    src = edge_index[0].astype(jnp.int32).reshape(_NW, _BLK_PER_TILE, _B)
    dst = edge_index[1].astype(jnp.int32).reshape(_NW, _BLK_PER_TILE, _B)

    We_s = We[:_DF]
    We_d = We[_DF:2 * _DF]
    We_e = We[2 * _DF:2 * _DF + _DE]
    We_u = We[2 * _DF + _DE:]
    Wn_a = Wn[:_DE]
    Wn_n = Wn[_DE:_DE + _DF]
    Wn_u = Wn[_DE + _DF:]
    Wg_n = Wg[:_DF]
    Wg_e = Wg[_DF:_DF + _DE]
    Wg_u = Wg[_DF + _DE:]
    be2 = be.reshape(1, _DE)
    bn2 = bn.reshape(1, _DF)
    bg2 = bg.reshape(1, _DU)

    ngrid = _N // _NBLK
    ps, pd = pl.pallas_call(
        _proj_body,
        grid=(ngrid,),
        in_specs=[
            pl.BlockSpec((_NBLK, _DF), lambda i: (i, 0)),
            pl.BlockSpec((_DF, _DE), lambda i: (0, 0)),
            pl.BlockSpec((_DF, _DE), lambda i: (0, 0)),
        ],
        out_specs=[
            pl.BlockSpec((_NBLK, _DE), lambda i: (i, 0)),
            pl.BlockSpec((_NBLK, _DE), lambda i: (i, 0)),
        ],
        out_shape=[
            jax.ShapeDtypeStruct((_N, _DE), F32),
            jax.ShapeDtypeStruct((_N, _DE), F32),
        ],
    )(nf, We_s, We_d)

    egrid = _E // _EBLK
    q = pl.pallas_call(
        _q_body,
        grid=(egrid,),
        in_specs=[
            pl.BlockSpec((_EBLK, _DE), lambda i: (i, 0)),
            pl.BlockSpec((_DE, _DE), lambda i: (0, 0)),
            pl.BlockSpec((1, _DU), lambda i: (0, 0)),
            pl.BlockSpec((_DU, _DE), lambda i: (0, 0)),
            pl.BlockSpec((1, _DE), lambda i: (0, 0)),
        ],
        out_specs=pl.BlockSpec((_EBLK, _DE), lambda i: (i, 0)),
        out_shape=jax.ShapeDtypeStruct((_E, _DE), F32),
    )(ef, We_e, u, We_u, be2)

    oef, aggp, degp, esum = _sc_edge(ps, pd, q, ef, src, dst)
    esum = esum.reshape(_NW, _DE)

    onf, nsum = pl.pallas_call(
        _node_body,
        grid=(ngrid,),
        in_specs=[
            pl.BlockSpec((_NBLK, _DF), lambda i: (i, 0)),
            pl.BlockSpec((_NC, _NBLK, _DE), lambda i: (0, i, 0)),
            pl.BlockSpec((_NC, _NBLK, _DE), lambda i: (0, i, 0)),
            pl.BlockSpec((_DE, _DF), lambda i: (0, 0)),
            pl.BlockSpec((_DF, _DF), lambda i: (0, 0)),
            pl.BlockSpec((1, _DU), lambda i: (0, 0)),
            pl.BlockSpec((_DU, _DF), lambda i: (0, 0)),
            pl.BlockSpec((1, _DF), lambda i: (0, 0)),
        ],
        out_specs=[
            pl.BlockSpec((_NBLK, _DF), lambda i: (i, 0)),
            pl.BlockSpec((1, _DF), lambda i: (0, 0)),
        ],
        out_shape=[
            jax.ShapeDtypeStruct((_N, _DF), F32),
            jax.ShapeDtypeStruct((1, _DF), F32),
        ],
    )(nf, aggp, degp, Wn_a, Wn_n, u, Wn_u, bn2)

    ou = pl.pallas_call(
        _glob_body,
        out_shape=jax.ShapeDtypeStruct((1, _DU), F32),
    )(nsum, esum, u, Wg_n, Wg_e, Wg_u, bg2)

    return onf, oef, ou


# speedup vs baseline: 6.7102x; 1.6212x over previous
import jax
import jax.numpy as jnp
from jax import lax
from jax.experimental import pallas as pl
from jax.experimental.pallas import tpu as pltpu
from jax.experimental.pallas import tpu_sc as plsc

F32 = jnp.float32

_N = 10000
_E = 320000
_DF = 128
_DE = 16
_DU = 32

_NC = 2
_NS = 16
_NW = _NC * _NS
_B = 128
_G = 4
_SB = _B * _G
_NBLK = _E // _SB
_EC = _E // _B
_ZROWS = _N // _NS


def _sc_edge_body(ps_hbm, pd_hbm, qt_hbm, src_hbm, dst_hbm,
                  mt_hbm, agg_hbm,
                  idxs, idxd, gs, gd, mb, mtb, zb,
                  agg_sh, sem_a, sem_c, sem_d):
    c = lax.axis_index("c")
    s = lax.axis_index("s")
    tid = c * _NS + s

    onerow = jnp.ones((_DE,), F32)
    zrow = jnp.zeros((_DE,), F32)

    def _ones_fill(i, carry):
        mb[i, 1] = onerow
        return carry

    lax.fori_loop(0, _SB, _ones_fill, 0)

    def _zero_fill(i, carry):
        zb[i, 0] = zrow
        zb[i, 1] = zrow
        return carry

    lax.fori_loop(0, _ZROWS, _zero_fill, 0)
    pltpu.sync_copy(zb, agg_sh.at[pl.ds(s * _ZROWS, _ZROWS)])
    plsc.subcore_barrier()

    nblk = jnp.where(tid < (_NBLK - _NW * (_NBLK // _NW)),
                     _NBLK // _NW + 1, _NBLK // _NW)
    i16 = lax.iota(jnp.int32, 16)

    def _block(i, carry):
        bid = tid + _NW * i
        col0 = bid * _G

        pltpu.sync_copy(src_hbm.at[bid], idxs)
        pltpu.sync_copy(dst_hbm.at[bid], idxd)

        cps = [pltpu.async_copy(
            qt_hbm.at[:, pl.ds(col0, _G), :], mtb, sem_a)]
        for k in range(_G):
            cps.append(pltpu.async_copy(
                ps_hbm.at[idxs.at[k]], gs.at[pl.ds(k * _B, _B)], sem_c))
            cps.append(pltpu.async_copy(
                pd_hbm.at[idxd.at[k]], gd.at[pl.ds(k * _B, _B)], sem_d))
        for cp in cps:
            cp.wait()

        for k in range(_G):
            kf = jnp.full((16,), k, jnp.int32)

            def _row(j, a, kf=kf, k=k):
                jf = jnp.full((16,), j, jnp.int32)
                e = k * _B + j
                m = plsc.load_gather(mtb, [i16, kf, jf]) + gs[e] + gd[e]
                plsc.store_scatter(mtb, [i16, kf, jf], m)
                mb[e, 0] = m
                return a

            lax.fori_loop(0, _B, _row, 0)

        wps = [pltpu.async_copy(
            mtb, mt_hbm.at[:, pl.ds(col0, _G), :], sem_a)]
        for k in range(_G):
            wps.append(pltpu.async_copy(
                mb.at[pl.ds(k * _B, _B)], agg_sh.at[idxd.at[k]], sem_c,
                add=True))
        for cp in wps:
            cp.wait()
        return carry

    lax.fori_loop(0, nblk, _block, 0)

    plsc.subcore_barrier()

    @pl.when(s == 0)
    def _copy_out():
        pltpu.sync_copy(agg_sh, agg_hbm.at[c])


_sc_mesh = plsc.VectorSubcoreMesh(
    core_axis_name="c", subcore_axis_name="s",
    num_cores=_NC, num_subcores=_NS)

_sc_edge = pl.kernel(
    _sc_edge_body,
    out_type=(
        jax.ShapeDtypeStruct((16, _EC, _B), F32),
        jax.ShapeDtypeStruct((_NC, _N, 2, _DE), F32),
    ),
    mesh=_sc_mesh,
    scratch_types=[
        pltpu.VMEM((_G, _B), jnp.int32),
        pltpu.VMEM((_G, _B), jnp.int32),
        pltpu.VMEM((_SB, _DE), F32),
        pltpu.VMEM((_SB, _DE), F32),
        pltpu.VMEM((_SB, 2, _DE), F32),
        pltpu.VMEM((16, _G, _B), F32),
        pltpu.VMEM((_ZROWS, 2, _DE), F32),
        pltpu.VMEM_SHARED((_N, 2, _DE), F32),
        pltpu.SemaphoreType.DMA,
        pltpu.SemaphoreType.DMA,
        pltpu.SemaphoreType.DMA,
    ],
    compiler_params=pltpu.CompilerParams(use_tc_tiling_on_sc=False,
                                         needs_layout_passes=False),
)


_NBLKR = 400
_TBLK = 32000


def _proj_body(nf_ref, ws_ref, wd_ref, ps_ref, pd_ref):
    x = nf_ref[...]
    ps_ref[...] = jnp.dot(x, ws_ref[...], preferred_element_type=F32)
    pd_ref[...] = jnp.dot(x, wd_ref[...], preferred_element_type=F32)


def _qt_body(eft_ref, weet_ref, weu_ref, u_ref, bet_ref, qt_ref):
    cst = lax.dot_general(weu_ref[...], u_ref[...], (((0,), (1,)), ((), ())),
                          preferred_element_type=F32) + bet_ref[...]
    qt_ref[...] = jnp.dot(weet_ref[...], eft_ref[...],
                          preferred_element_type=F32) + cst


def _efin_body(mt_ref, eft_ref, oef_ref, es_ref):
    m = mt_ref[...]
    oef_ref[...] = m + eft_ref[...]

    @pl.when(pl.program_id(0) == 0)
    def _():
        es_ref[...] = jnp.zeros_like(es_ref)

    es_ref[...] += jnp.sum(m, axis=1, keepdims=True)


def _node_body(nf_ref, ad_ref, wa_ref, wn_ref, u_ref, wnu_ref,
               bn_ref, out_ref, ns_ref):
    agg = ad_ref[0, :, 0, :] + ad_ref[1, :, 0, :]
    deg = ad_ref[0, :, 1, :] + ad_ref[1, :, 1, :]
    aggm = agg / jnp.maximum(deg, 1.0)
    cst = jnp.dot(u_ref[...], wnu_ref[...], preferred_element_type=F32) \
        + bn_ref[...]
    pre = (jnp.dot(aggm, wa_ref[...], preferred_element_type=F32)
           + jnp.dot(nf_ref[...], wn_ref[...], preferred_element_type=F32)
           + cst)
    out_ref[...] = pre + nf_ref[...]

    @pl.when(pl.program_id(0) == 0)
    def _():
        ns_ref[...] = jnp.zeros_like(ns_ref)

    ns_ref[...] += jnp.sum(pre, axis=0, keepdims=True)


def _glob_body(ns_ref, est_ref, u_ref, wgn_ref, wge_ref, wgu_ref, bg_ref,
               ou_ref):
    nr = ns_ref[...] * (1.0 / _N)
    erc = lax.dot_general(est_ref[...], wge_ref[...],
                          (((0,), (0,)), ((), ())),
                          preferred_element_type=F32) * (1.0 / _E)
    ou_ref[...] = (jnp.dot(nr, wgn_ref[...], preferred_element_type=F32)
                   + erc
                   + jnp.dot(u_ref[...], wgu_ref[...],
                             preferred_element_type=F32)
                   + bg_ref[...] + u_ref[...])


def kernel(nf, ef, u, edge_index, We, be, Wn, bn, Wg, bg):
    assert nf.shape == (_N, _DF) and ef.shape == (_E, _DE)
    assert u.shape == (1, _DU) and edge_index.shape == (2, _E)

    src = edge_index[0].astype(jnp.int32).reshape(_NBLK, _G, _B)
    dst = edge_index[1].astype(jnp.int32).reshape(_NBLK, _G, _B)

    We_s = We[:_DF]
    We_d = We[_DF:2 * _DF]
    We_e = We[2 * _DF:2 * _DF + _DE]
    We_u = We[2 * _DF + _DE:]
    wee_t = We_e.T
    be_t = be.reshape(_DE, 1)
    Wn_a = Wn[:_DE]
    Wn_n = Wn[_DE:_DE + _DF]
    Wn_u = Wn[_DE + _DF:]
    Wg_n = Wg[:_DF]
    Wg_e = Wg[_DF:_DF + _DE]
    Wg_u = Wg[_DF + _DE:]
    bn2 = bn.reshape(1, _DF)
    bg2 = bg.reshape(1, _DU)

    ef_t = ef.T

    ngrid = _N // _NBLKR
    ps, pd = pl.pallas_call(
        _proj_body,
        grid=(ngrid,),
        in_specs=[
            pl.BlockSpec((_NBLKR, _DF), lambda i: (i, 0)),
            pl.BlockSpec((_DF, _DE), lambda i: (0, 0)),
            pl.BlockSpec((_DF, _DE), lambda i: (0, 0)),
        ],
        out_specs=[
            pl.BlockSpec((_NBLKR, _DE), lambda i: (i, 0)),
            pl.BlockSpec((_NBLKR, _DE), lambda i: (i, 0)),
        ],
        out_shape=[
            jax.ShapeDtypeStruct((_N, _DE), F32),
            jax.ShapeDtypeStruct((_N, _DE), F32),
        ],
    )(nf, We_s, We_d)

    qt = pl.pallas_call(
        _qt_body,
        grid=(_E // _TBLK,),
        in_specs=[
            pl.BlockSpec((16, _TBLK), lambda i: (0, i)),
            pl.BlockSpec((_DE, _DE), lambda i: (0, 0)),
            pl.BlockSpec((_DU, _DE), lambda i: (0, 0)),
            pl.BlockSpec((1, _DU), lambda i: (0, 0)),
            pl.BlockSpec((_DE, 1), lambda i: (0, 0)),
        ],
        out_specs=pl.BlockSpec((16, _TBLK), lambda i: (0, i)),
        out_shape=jax.ShapeDtypeStruct((16, _E), F32),
    )(ef_t, wee_t, We_u, u, be_t)

    mt3, aggdeg = _sc_edge(ps, pd, qt.reshape(16, _EC, _B), src, dst)
    mt = mt3.reshape(16, _E)

    oef_t, esum = pl.pallas_call(
        _efin_body,
        grid=(_E // _TBLK,),
        in_specs=[
            pl.BlockSpec((16, _TBLK), lambda i: (0, i)),
            pl.BlockSpec((16, _TBLK), lambda i: (0, i)),
        ],
        out_specs=[
            pl.BlockSpec((16, _TBLK), lambda i: (0, i)),
            pl.BlockSpec((16, 1), lambda i: (0, 0)),
        ],
        out_shape=[
            jax.ShapeDtypeStruct((16, _E), F32),
            jax.ShapeDtypeStruct((16, 1), F32),
        ],
    )(mt, ef_t)

    onf, nsum = pl.pallas_call(
        _node_body,
        grid=(ngrid,),
        in_specs=[
            pl.BlockSpec((_NBLKR, _DF), lambda i: (i, 0)),
            pl.BlockSpec((_NC, _NBLKR, 2, _DE), lambda i: (0, i, 0, 0)),
            pl.BlockSpec((_DE, _DF), lambda i: (0, 0)),
            pl.BlockSpec((_DF, _DF), lambda i: (0, 0)),
            pl.BlockSpec((1, _DU), lambda i: (0, 0)),
            pl.BlockSpec((_DU, _DF), lambda i: (0, 0)),
            pl.BlockSpec((1, _DF), lambda i: (0, 0)),
        ],
        out_specs=[
            pl.BlockSpec((_NBLKR, _DF), lambda i: (i, 0)),
            pl.BlockSpec((1, _DF), lambda i: (0, 0)),
        ],
        out_shape=[
            jax.ShapeDtypeStruct((_N, _DF), F32),
            jax.ShapeDtypeStruct((1, _DF), F32),
        ],
    )(nf, aggdeg, Wn_a, Wn_n, u, Wn_u, bn2)

    ou = pl.pallas_call(
        _glob_body,
        out_shape=jax.ShapeDtypeStruct((1, _DU), F32),
    )(nsum, esum, u, Wg_n, Wg_e, Wg_u, bg2)

    return onf, oef_t.T, ou


# speedup vs baseline: 7.2201x; 1.0760x over previous
import jax
import jax.numpy as jnp
from jax import lax
from jax.experimental import pallas as pl
from jax.experimental.pallas import tpu as pltpu
from jax.experimental.pallas import tpu_sc as plsc

F32 = jnp.float32

_N = 10000
_E = 320000
_DF = 128
_DE = 16
_DU = 32

_NC = 2
_NS = 16
_NW = _NC * _NS
_B = 128
_G = 5
_SB = _B * _G
_NBLK = _E // _SB
_EC = _E // _B
_ZROWS = _N // _NS


def _sc_edge_body(ps_hbm, pd_hbm, qt_hbm, src_hbm, dst_hbm,
                  mt_hbm, agg_hbm, deg_hbm,
                  idxs, idxd, gs, gd, mb, mtb, ones_v, zb,
                  agg_sh, deg_sh, sem_a, sem_c, sem_d, sem_i):
    c = lax.axis_index("c")
    s = lax.axis_index("s")
    tid = c * _NS + s

    onerow = jnp.ones((_DE,), F32)
    zrow = jnp.zeros((_DE,), F32)

    def _ones_fill(i, carry):
        ones_v[i] = onerow
        return carry

    lax.fori_loop(0, _B, _ones_fill, 0)

    def _zero_fill(i, carry):
        zb[i] = zrow
        return carry

    lax.fori_loop(0, _ZROWS, _zero_fill, 0)
    pltpu.sync_copy(zb, agg_sh.at[pl.ds(s * _ZROWS, _ZROWS)])
    pltpu.sync_copy(zb, deg_sh.at[pl.ds(s * _ZROWS, _ZROWS)])
    plsc.subcore_barrier()

    nblk = jnp.where(tid < (_NBLK - _NW * (_NBLK // _NW)),
                     _NBLK // _NW + 1, _NBLK // _NW)
    i16 = lax.iota(jnp.int32, 16)

    def _block(i, carry):
        bid = tid + _NW * i
        col0 = bid * _G

        i1 = pltpu.async_copy(src_hbm.at[bid], idxs, sem_i)
        i2 = pltpu.async_copy(dst_hbm.at[bid], idxd, sem_i)
        cps = [pltpu.async_copy(
            qt_hbm.at[:, pl.ds(col0, _G), :], mtb, sem_a)]
        i1.wait()
        i2.wait()
        for k in range(_G):
            cps.append(pltpu.async_copy(
                ps_hbm.at[idxs.at[k]], gs.at[pl.ds(k * _B, _B)], sem_c))
            cps.append(pltpu.async_copy(
                pd_hbm.at[idxd.at[k]], gd.at[pl.ds(k * _B, _B)], sem_d))
        for cp in cps:
            cp.wait()

        for k in range(_G):
            kf = jnp.full((16,), k, jnp.int32)

            def _row(j, a, kf=kf, k=k):
                jf = jnp.full((16,), j, jnp.int32)
                e = k * _B + j
                m = plsc.load_gather(mtb, [i16, kf, jf]) + gs[e] + gd[e]
                plsc.store_scatter(mtb, [i16, kf, jf], m)
                mb[e] = m
                return a

            lax.fori_loop(0, _B, _row, 0)

        wps = [pltpu.async_copy(
            mtb, mt_hbm.at[:, pl.ds(col0, _G), :], sem_a)]
        for k in range(_G):
            wps.append(pltpu.async_copy(
                mb.at[pl.ds(k * _B, _B)], agg_sh.at[idxd.at[k]], sem_c,
                add=True))
            wps.append(pltpu.async_copy(
                ones_v, deg_sh.at[idxd.at[k]], sem_d, add=True))
        for cp in wps:
            cp.wait()
        return carry

    lax.fori_loop(0, nblk, _block, 0)

    plsc.subcore_barrier()

    @pl.when(s == 0)
    def _copy_out():
        pltpu.sync_copy(agg_sh, agg_hbm.at[c])
        pltpu.sync_copy(deg_sh, deg_hbm.at[c])


_sc_mesh = plsc.VectorSubcoreMesh(
    core_axis_name="c", subcore_axis_name="s",
    num_cores=_NC, num_subcores=_NS)

_sc_edge = pl.kernel(
    _sc_edge_body,
    out_type=(
        jax.ShapeDtypeStruct((16, _EC, _B), F32),
        jax.ShapeDtypeStruct((_NC, _N, _DE), F32),
        jax.ShapeDtypeStruct((_NC, _N, _DE), F32),
    ),
    mesh=_sc_mesh,
    scratch_types=[
        pltpu.VMEM((_G, _B), jnp.int32),
        pltpu.VMEM((_G, _B), jnp.int32),
        pltpu.VMEM((_SB, _DE), F32),
        pltpu.VMEM((_SB, _DE), F32),
        pltpu.VMEM((_SB, _DE), F32),
        pltpu.VMEM((16, _G, _B), F32),
        pltpu.VMEM((_B, _DE), F32),
        pltpu.VMEM((_ZROWS, _DE), F32),
        pltpu.VMEM_SHARED((_N, _DE), F32),
        pltpu.VMEM_SHARED((_N, _DE), F32),
        pltpu.SemaphoreType.DMA,
        pltpu.SemaphoreType.DMA,
        pltpu.SemaphoreType.DMA,
        pltpu.SemaphoreType.DMA,
    ],
    compiler_params=pltpu.CompilerParams(use_tc_tiling_on_sc=False,
                                         needs_layout_passes=False),
)


_NBLKR = 400
_TBLK = 32000


def _proj_body(nf_ref, ws_ref, wd_ref, ps_ref, pd_ref):
    x = nf_ref[...]
    ps_ref[...] = jnp.dot(x, ws_ref[...], preferred_element_type=F32)
    pd_ref[...] = jnp.dot(x, wd_ref[...], preferred_element_type=F32)


def _qt_body(eft_ref, weet_ref, weu_ref, u_ref, bet_ref, qt_ref):
    cst = lax.dot_general(weu_ref[...], u_ref[...], (((0,), (1,)), ((), ())),
                          preferred_element_type=F32) + bet_ref[...]
    qt_ref[...] = jnp.dot(weet_ref[...], eft_ref[...],
                          preferred_element_type=F32) + cst


def _efin_body(mt_ref, eft_ref, oef_ref, es_ref):
    m = mt_ref[...]
    oef_ref[...] = m + eft_ref[...]

    @pl.when(pl.program_id(0) == 0)
    def _():
        es_ref[...] = jnp.zeros_like(es_ref)

    es_ref[...] += jnp.sum(m, axis=1, keepdims=True)


def _node_body(nf_ref, agg_ref, deg_ref, wa_ref, wn_ref, u_ref, wnu_ref,
               bn_ref, out_ref, ns_ref):
    agg = agg_ref[0] + agg_ref[1]
    deg = deg_ref[0] + deg_ref[1]
    aggm = agg / jnp.maximum(deg, 1.0)
    cst = jnp.dot(u_ref[...], wnu_ref[...], preferred_element_type=F32) \
        + bn_ref[...]
    pre = (jnp.dot(aggm, wa_ref[...], preferred_element_type=F32)
           + jnp.dot(nf_ref[...], wn_ref[...], preferred_element_type=F32)
           + cst)
    out_ref[...] = pre + nf_ref[...]

    @pl.when(pl.program_id(0) == 0)
    def _():
        ns_ref[...] = jnp.zeros_like(ns_ref)

    ns_ref[...] += jnp.sum(pre, axis=0, keepdims=True)


def _glob_body(ns_ref, est_ref, u_ref, wgn_ref, wge_ref, wgu_ref, bg_ref,
               ou_ref):
    nr = ns_ref[...] * (1.0 / _N)
    erc = lax.dot_general(est_ref[...], wge_ref[...],
                          (((0,), (0,)), ((), ())),
                          preferred_element_type=F32) * (1.0 / _E)
    ou_ref[...] = (jnp.dot(nr, wgn_ref[...], preferred_element_type=F32)
                   + erc
                   + jnp.dot(u_ref[...], wgu_ref[...],
                             preferred_element_type=F32)
                   + bg_ref[...] + u_ref[...])


def kernel(nf, ef, u, edge_index, We, be, Wn, bn, Wg, bg):
    assert nf.shape == (_N, _DF) and ef.shape == (_E, _DE)
    assert u.shape == (1, _DU) and edge_index.shape == (2, _E)

    src = edge_index[0].astype(jnp.int32).reshape(_NBLK, _G, _B)
    dst = edge_index[1].astype(jnp.int32).reshape(_NBLK, _G, _B)

    We_s = We[:_DF]
    We_d = We[_DF:2 * _DF]
    We_e = We[2 * _DF:2 * _DF + _DE]
    We_u = We[2 * _DF + _DE:]
    wee_t = We_e.T
    be_t = be.reshape(_DE, 1)
    Wn_a = Wn[:_DE]
    Wn_n = Wn[_DE:_DE + _DF]
    Wn_u = Wn[_DE + _DF:]
    Wg_n = Wg[:_DF]
    Wg_e = Wg[_DF:_DF + _DE]
    Wg_u = Wg[_DF + _DE:]
    bn2 = bn.reshape(1, _DF)
    bg2 = bg.reshape(1, _DU)

    ef_t = ef.T

    ngrid = _N // _NBLKR
    ps, pd = pl.pallas_call(
        _proj_body,
        grid=(ngrid,),
        in_specs=[
            pl.BlockSpec((_NBLKR, _DF), lambda i: (i, 0)),
            pl.BlockSpec((_DF, _DE), lambda i: (0, 0)),
            pl.BlockSpec((_DF, _DE), lambda i: (0, 0)),
        ],
        out_specs=[
            pl.BlockSpec((_NBLKR, _DE), lambda i: (i, 0)),
            pl.BlockSpec((_NBLKR, _DE), lambda i: (i, 0)),
        ],
        out_shape=[
            jax.ShapeDtypeStruct((_N, _DE), F32),
            jax.ShapeDtypeStruct((_N, _DE), F32),
        ],
    )(nf, We_s, We_d)

    qt = pl.pallas_call(
        _qt_body,
        grid=(_E // _TBLK,),
        in_specs=[
            pl.BlockSpec((16, _TBLK), lambda i: (0, i)),
            pl.BlockSpec((_DE, _DE), lambda i: (0, 0)),
            pl.BlockSpec((_DU, _DE), lambda i: (0, 0)),
            pl.BlockSpec((1, _DU), lambda i: (0, 0)),
            pl.BlockSpec((_DE, 1), lambda i: (0, 0)),
        ],
        out_specs=pl.BlockSpec((16, _TBLK), lambda i: (0, i)),
        out_shape=jax.ShapeDtypeStruct((16, _E), F32),
    )(ef_t, wee_t, We_u, u, be_t)

    mt3, aggp, degp = _sc_edge(ps, pd, qt.reshape(16, _EC, _B), src, dst)
    mt = mt3.reshape(16, _E)

    oef_t, esum = pl.pallas_call(
        _efin_body,
        grid=(_E // _TBLK,),
        in_specs=[
            pl.BlockSpec((16, _TBLK), lambda i: (0, i)),
            pl.BlockSpec((16, _TBLK), lambda i: (0, i)),
        ],
        out_specs=[
            pl.BlockSpec((16, _TBLK), lambda i: (0, i)),
            pl.BlockSpec((16, 1), lambda i: (0, 0)),
        ],
        out_shape=[
            jax.ShapeDtypeStruct((16, _E), F32),
            jax.ShapeDtypeStruct((16, 1), F32),
        ],
    )(mt, ef_t)

    onf, nsum = pl.pallas_call(
        _node_body,
        grid=(ngrid,),
        in_specs=[
            pl.BlockSpec((_NBLKR, _DF), lambda i: (i, 0)),
            pl.BlockSpec((_NC, _NBLKR, _DE), lambda i: (0, i, 0)),
            pl.BlockSpec((_NC, _NBLKR, _DE), lambda i: (0, i, 0)),
            pl.BlockSpec((_DE, _DF), lambda i: (0, 0)),
            pl.BlockSpec((_DF, _DF), lambda i: (0, 0)),
            pl.BlockSpec((1, _DU), lambda i: (0, 0)),
            pl.BlockSpec((_DU, _DF), lambda i: (0, 0)),
            pl.BlockSpec((1, _DF), lambda i: (0, 0)),
        ],
        out_specs=[
            pl.BlockSpec((_NBLKR, _DF), lambda i: (i, 0)),
            pl.BlockSpec((1, _DF), lambda i: (0, 0)),
        ],
        out_shape=[
            jax.ShapeDtypeStruct((_N, _DF), F32),
            jax.ShapeDtypeStruct((1, _DF), F32),
        ],
    )(nf, aggp, degp, Wn_a, Wn_n, u, Wn_u, bn2)

    ou = pl.pallas_call(
        _glob_body,
        out_shape=jax.ShapeDtypeStruct((1, _DU), F32),
    )(nsum, esum, u, Wg_n, Wg_e, Wg_u, bg2)

    return onf, oef_t.T, ou


# speedup vs baseline: 7.3165x; 1.0134x over previous
import jax
import jax.numpy as jnp
from jax import lax
from jax.experimental import pallas as pl
from jax.experimental.pallas import tpu as pltpu
from jax.experimental.pallas import tpu_sc as plsc

F32 = jnp.float32

_N = 10000
_E = 320000
_DF = 128
_DE = 16
_DU = 32

_NC = 2
_NS = 16
_NW = _NC * _NS
_B = 128
_G = 10
_SB = _B * _G
_NBLK = _E // _SB
_EC = _E // _B
_ZROWS = _N // _NS


def _sc_edge_body(ps_hbm, pd_hbm, qt_hbm, src_hbm, dst_hbm,
                  mt_hbm, agg_hbm, deg_hbm,
                  idxs, idxd, gs, gd, mb, mtb, ones_v, zb,
                  agg_sh, deg_sh, sem_a, sem_c, sem_d, sem_i):
    c = lax.axis_index("c")
    s = lax.axis_index("s")
    tid = c * _NS + s

    onerow = jnp.ones((_DE,), F32)
    zrow = jnp.zeros((_DE,), F32)

    def _ones_fill(i, carry):
        ones_v[i] = onerow
        return carry

    lax.fori_loop(0, _B, _ones_fill, 0)

    def _zero_fill(i, carry):
        zb[i] = zrow
        return carry

    lax.fori_loop(0, _ZROWS, _zero_fill, 0)
    pltpu.sync_copy(zb, agg_sh.at[pl.ds(s * _ZROWS, _ZROWS)])
    pltpu.sync_copy(zb, deg_sh.at[pl.ds(s * _ZROWS, _ZROWS)])
    plsc.subcore_barrier()

    nblk = jnp.where(tid < (_NBLK - _NW * (_NBLK // _NW)),
                     _NBLK // _NW + 1, _NBLK // _NW)
    i16 = lax.iota(jnp.int32, 16)

    def _block(i, carry):
        bid = tid + _NW * i
        col0 = bid * _G

        i1 = pltpu.async_copy(src_hbm.at[bid], idxs, sem_i)
        i2 = pltpu.async_copy(dst_hbm.at[bid], idxd, sem_i)
        cps = [pltpu.async_copy(
            qt_hbm.at[:, pl.ds(col0, _G), :], mtb, sem_a)]
        i1.wait()
        i2.wait()
        for k in range(_G):
            cps.append(pltpu.async_copy(
                ps_hbm.at[idxs.at[k]], gs.at[pl.ds(k * _B, _B)], sem_c))
            cps.append(pltpu.async_copy(
                pd_hbm.at[idxd.at[k]], gd.at[pl.ds(k * _B, _B)], sem_d))
        for cp in cps:
            cp.wait()

        for k in range(_G):
            kf = jnp.full((16,), k, jnp.int32)

            def _row(j, a, kf=kf, k=k):
                jf = jnp.full((16,), j, jnp.int32)
                e = k * _B + j
                m = plsc.load_gather(mtb, [i16, kf, jf]) + gs[e] + gd[e]
                plsc.store_scatter(mtb, [i16, kf, jf], m)
                mb[e] = m
                return a

            lax.fori_loop(0, _B, _row, 0)

        wps = [pltpu.async_copy(
            mtb, mt_hbm.at[:, pl.ds(col0, _G), :], sem_a)]
        for k in range(_G):
            wps.append(pltpu.async_copy(
                mb.at[pl.ds(k * _B, _B)], agg_sh.at[idxd.at[k]], sem_c,
                add=True))
            wps.append(pltpu.async_copy(
                ones_v, deg_sh.at[idxd.at[k]], sem_d, add=True))
        for cp in wps:
            cp.wait()
        return carry

    lax.fori_loop(0, nblk, _block, 0)

    plsc.subcore_barrier()

    @pl.when(s == 0)
    def _copy_out():
        pltpu.sync_copy(agg_sh, agg_hbm.at[c])
        pltpu.sync_copy(deg_sh, deg_hbm.at[c])


_sc_mesh = plsc.VectorSubcoreMesh(
    core_axis_name="c", subcore_axis_name="s",
    num_cores=_NC, num_subcores=_NS)

_sc_edge = pl.kernel(
    _sc_edge_body,
    out_type=(
        jax.ShapeDtypeStruct((16, _EC, _B), F32),
        jax.ShapeDtypeStruct((_NC, _N, _DE), F32),
        jax.ShapeDtypeStruct((_NC, _N, _DE), F32),
    ),
    mesh=_sc_mesh,
    scratch_types=[
        pltpu.VMEM((_G, _B), jnp.int32),
        pltpu.VMEM((_G, _B), jnp.int32),
        pltpu.VMEM((_SB, _DE), F32),
        pltpu.VMEM((_SB, _DE), F32),
        pltpu.VMEM((_SB, _DE), F32),
        pltpu.VMEM((16, _G, _B), F32),
        pltpu.VMEM((_B, _DE), F32),
        pltpu.VMEM((_ZROWS, _DE), F32),
        pltpu.VMEM_SHARED((_N, _DE), F32),
        pltpu.VMEM_SHARED((_N, _DE), F32),
        pltpu.SemaphoreType.DMA,
        pltpu.SemaphoreType.DMA,
        pltpu.SemaphoreType.DMA,
        pltpu.SemaphoreType.DMA,
    ],
    compiler_params=pltpu.CompilerParams(use_tc_tiling_on_sc=False,
                                         needs_layout_passes=False),
)


_NBLKR = 400
_TBLK = 32000


def _proj_body(nf_ref, ws_ref, wd_ref, ps_ref, pd_ref):
    x = nf_ref[...]
    ps_ref[...] = jnp.dot(x, ws_ref[...], preferred_element_type=F32)
    pd_ref[...] = jnp.dot(x, wd_ref[...], preferred_element_type=F32)


def _qt_body(eft_ref, weet_ref, weu_ref, u_ref, bet_ref, qt_ref):
    cst = lax.dot_general(weu_ref[...], u_ref[...], (((0,), (1,)), ((), ())),
                          preferred_element_type=F32) + bet_ref[...]
    qt_ref[...] = jnp.dot(weet_ref[...], eft_ref[...],
                          preferred_element_type=F32) + cst


def _efin_body(mt_ref, eft_ref, oef_ref, es_ref):
    m = mt_ref[...]
    oef_ref[...] = m + eft_ref[...]

    @pl.when(pl.program_id(0) == 0)
    def _():
        es_ref[...] = jnp.zeros_like(es_ref)

    es_ref[...] += jnp.sum(m, axis=1, keepdims=True)


def _node_body(nf_ref, agg_ref, deg_ref, wa_ref, wn_ref, u_ref, wnu_ref,
               bn_ref, out_ref, ns_ref):
    agg = agg_ref[0] + agg_ref[1]
    deg = deg_ref[0] + deg_ref[1]
    aggm = agg / jnp.maximum(deg, 1.0)
    cst = jnp.dot(u_ref[...], wnu_ref[...], preferred_element_type=F32) \
        + bn_ref[...]
    pre = (jnp.dot(aggm, wa_ref[...], preferred_element_type=F32)
           + jnp.dot(nf_ref[...], wn_ref[...], preferred_element_type=F32)
           + cst)
    out_ref[...] = pre + nf_ref[...]

    @pl.when(pl.program_id(0) == 0)
    def _():
        ns_ref[...] = jnp.zeros_like(ns_ref)

    ns_ref[...] += jnp.sum(pre, axis=0, keepdims=True)


def _glob_body(ns_ref, est_ref, u_ref, wgn_ref, wge_ref, wgu_ref, bg_ref,
               ou_ref):
    nr = ns_ref[...] * (1.0 / _N)
    erc = lax.dot_general(est_ref[...], wge_ref[...],
                          (((0,), (0,)), ((), ())),
                          preferred_element_type=F32) * (1.0 / _E)
    ou_ref[...] = (jnp.dot(nr, wgn_ref[...], preferred_element_type=F32)
                   + erc
                   + jnp.dot(u_ref[...], wgu_ref[...],
                             preferred_element_type=F32)
                   + bg_ref[...] + u_ref[...])


def kernel(nf, ef, u, edge_index, We, be, Wn, bn, Wg, bg):
    assert nf.shape == (_N, _DF) and ef.shape == (_E, _DE)
    assert u.shape == (1, _DU) and edge_index.shape == (2, _E)

    src = edge_index[0].astype(jnp.int32).reshape(_NBLK, _G, _B)
    dst = edge_index[1].astype(jnp.int32).reshape(_NBLK, _G, _B)

    We_s = We[:_DF]
    We_d = We[_DF:2 * _DF]
    We_e = We[2 * _DF:2 * _DF + _DE]
    We_u = We[2 * _DF + _DE:]
    wee_t = We_e.T
    be_t = be.reshape(_DE, 1)
    Wn_a = Wn[:_DE]
    Wn_n = Wn[_DE:_DE + _DF]
    Wn_u = Wn[_DE + _DF:]
    Wg_n = Wg[:_DF]
    Wg_e = Wg[_DF:_DF + _DE]
    Wg_u = Wg[_DF + _DE:]
    bn2 = bn.reshape(1, _DF)
    bg2 = bg.reshape(1, _DU)

    ef_t = ef.T

    ngrid = _N // _NBLKR
    ps, pd = pl.pallas_call(
        _proj_body,
        grid=(ngrid,),
        in_specs=[
            pl.BlockSpec((_NBLKR, _DF), lambda i: (i, 0)),
            pl.BlockSpec((_DF, _DE), lambda i: (0, 0)),
            pl.BlockSpec((_DF, _DE), lambda i: (0, 0)),
        ],
        out_specs=[
            pl.BlockSpec((_NBLKR, _DE), lambda i: (i, 0)),
            pl.BlockSpec((_NBLKR, _DE), lambda i: (i, 0)),
        ],
        out_shape=[
            jax.ShapeDtypeStruct((_N, _DE), F32),
            jax.ShapeDtypeStruct((_N, _DE), F32),
        ],
    )(nf, We_s, We_d)

    qt = pl.pallas_call(
        _qt_body,
        grid=(_E // _TBLK,),
        in_specs=[
            pl.BlockSpec((16, _TBLK), lambda i: (0, i)),
            pl.BlockSpec((_DE, _DE), lambda i: (0, 0)),
            pl.BlockSpec((_DU, _DE), lambda i: (0, 0)),
            pl.BlockSpec((1, _DU), lambda i: (0, 0)),
            pl.BlockSpec((_DE, 1), lambda i: (0, 0)),
        ],
        out_specs=pl.BlockSpec((16, _TBLK), lambda i: (0, i)),
        out_shape=jax.ShapeDtypeStruct((16, _E), F32),
    )(ef_t, wee_t, We_u, u, be_t)

    mt3, aggp, degp = _sc_edge(ps, pd, qt.reshape(16, _EC, _B), src, dst)
    mt = mt3.reshape(16, _E)

    oef_t, esum = pl.pallas_call(
        _efin_body,
        grid=(_E // _TBLK,),
        in_specs=[
            pl.BlockSpec((16, _TBLK), lambda i: (0, i)),
            pl.BlockSpec((16, _TBLK), lambda i: (0, i)),
        ],
        out_specs=[
            pl.BlockSpec((16, _TBLK), lambda i: (0, i)),
            pl.BlockSpec((16, 1), lambda i: (0, 0)),
        ],
        out_shape=[
            jax.ShapeDtypeStruct((16, _E), F32),
            jax.ShapeDtypeStruct((16, 1), F32),
        ],
    )(mt, ef_t)

    onf, nsum = pl.pallas_call(
        _node_body,
        grid=(ngrid,),
        in_specs=[
            pl.BlockSpec((_NBLKR, _DF), lambda i: (i, 0)),
            pl.BlockSpec((_NC, _NBLKR, _DE), lambda i: (0, i, 0)),
            pl.BlockSpec((_NC, _NBLKR, _DE), lambda i: (0, i, 0)),
            pl.BlockSpec((_DE, _DF), lambda i: (0, 0)),
            pl.BlockSpec((_DF, _DF), lambda i: (0, 0)),
            pl.BlockSpec((1, _DU), lambda i: (0, 0)),
            pl.BlockSpec((_DU, _DF), lambda i: (0, 0)),
            pl.BlockSpec((1, _DF), lambda i: (0, 0)),
        ],
        out_specs=[
            pl.BlockSpec((_NBLKR, _DF), lambda i: (i, 0)),
            pl.BlockSpec((1, _DF), lambda i: (0, 0)),
        ],
        out_shape=[
            jax.ShapeDtypeStruct((_N, _DF), F32),
            jax.ShapeDtypeStruct((1, _DF), F32),
        ],
    )(nf, aggp, degp, Wn_a, Wn_n, u, Wn_u, bn2)

    ou = pl.pallas_call(
        _glob_body,
        out_shape=jax.ShapeDtypeStruct((1, _DU), F32),
    )(nsum, esum, u, Wg_n, Wg_e, Wg_u, bg2)

    return onf, oef_t.T, ou
